# initial kernel scaffold (unmeasured)
import jax
import jax.numpy as jnp
from jax import lax
from jax.experimental import pallas as pl
from jax.experimental.pallas import tpu as pltpu

N_DEV = 16


def kernel(x, w_mat, scale_x, scale_w):
    k_glob, m_per = x.shape
    _, n = w_mat.shape
    m_blk = k_glob // N_DEV

    def body(x_ref, w_ref, sx_ref, sw_ref, out_ref, comm_ref,
             send_sems, recv_sems):
        p = lax.axis_index("i")

        rdmas = []
        for j in range(1, N_DEV):
            d = (p + j) % N_DEV
            rdma = pltpu.make_async_remote_copy(
                src_ref=x_ref.at[pl.ds(d * m_blk, m_blk), :],
                dst_ref=comm_ref.at[j],
                send_sem=send_sems.at[j],
                recv_sem=recv_sems.at[j],
                device_id=(d,),
                device_id_type=pl.DeviceIdType.MESH,
            )
            rdma.start()
            rdmas.append(rdma)

        x_own = x_ref[pl.ds(p * m_blk, m_blk), :]
        w_own = w_ref[pl.ds(p * m_per, m_per), :]
        acc = lax.dot_general(
            x_own, w_own, (((1,), (0,)), ((), ())),
            preferred_element_type=jnp.int32,
        )

        for j in range(1, N_DEV):
            rdmas[j - 1].wait_recv()
            src = (p - j) % N_DEV
            w_blk = w_ref[pl.ds(src * m_per, m_per), :]
            acc += lax.dot_general(
                comm_ref[j], w_blk, (((1,), (0,)), ((), ())),
                preferred_element_type=jnp.int32,
            )

        for j in range(1, N_DEV):
            rdmas[j - 1].wait_send()

        scale = sx_ref[0] * sw_ref[0]
        y = acc.astype(jnp.float32) * scale
        out_ref[:, :] = jnp.maximum(y, 0.0)

    return pl.pallas_call(
        body,
        out_shape=jax.ShapeDtypeStruct((m_blk, n), jnp.float32),
        in_specs=[
            pl.BlockSpec(memory_space=pltpu.VMEM),
            pl.BlockSpec(memory_space=pltpu.VMEM),
            pl.BlockSpec(memory_space=pltpu.SMEM),
            pl.BlockSpec(memory_space=pltpu.SMEM),
        ],
        out_specs=pl.BlockSpec(memory_space=pltpu.VMEM),
        scratch_shapes=[
            pltpu.VMEM((N_DEV, m_blk, m_per), jnp.int8),
            pltpu.SemaphoreType.DMA((N_DEV,)),
            pltpu.SemaphoreType.DMA((N_DEV,)),
        ],
        compiler_params=pltpu.CompilerParams(collective_id=0),
    )(x, w_mat, scale_x, scale_w)


# baseline (device time: 11689 ns/iter reference)
import jax
import jax.numpy as jnp
from jax import lax
from jax.experimental import pallas as pl
from jax.experimental.pallas import tpu as pltpu

N_DEV = 16


def kernel(x, w_mat, scale_x, scale_w):
    k_glob, m_per = x.shape
    _, n = w_mat.shape
    m_blk = k_glob // N_DEV

    def body(x_ref, w_ref, sx_ref, sw_ref, out_ref, comm_ref,
             send_sems, recv_sems):
        p = lax.axis_index("i")

        rdmas = []
        for j in range(1, N_DEV):
            d = (p + j) % N_DEV
            rdma = pltpu.make_async_remote_copy(
                src_ref=x_ref.at[pl.ds(d * m_blk, m_blk), :],
                dst_ref=comm_ref.at[j],
                send_sem=send_sems.at[j],
                recv_sem=recv_sems.at[j],
                device_id=(d,),
                device_id_type=pl.DeviceIdType.MESH,
            )
            rdma.start()
            rdmas.append(rdma)

        x_own = x_ref[pl.ds(p * m_blk, m_blk), :]
        w_own = w_ref[pl.ds(p * m_per, m_per), :]
        acc = lax.dot_general(
            x_own, w_own, (((1,), (0,)), ((), ())),
            preferred_element_type=jnp.int32,
        )

        for j in range(1, N_DEV):
            rdmas[j - 1].wait_recv()
            src = (p - j) % N_DEV
            w_blk = w_ref[pl.ds(src * m_per, m_per), :]
            acc += lax.dot_general(
                comm_ref[j], w_blk, (((1,), (0,)), ((), ())),
                preferred_element_type=jnp.int32,
            )

        for j in range(1, N_DEV):
            rdmas[j - 1].wait_send()

        scale = sx_ref[0] * sw_ref[0]
        y = acc.astype(jnp.float32) * scale
        out_ref[:, :] = jnp.maximum(y, 0.0)

    return pl.pallas_call(
        body,
        out_shape=jax.ShapeDtypeStruct((m_blk, n), jnp.float32),
        in_specs=[
            pl.BlockSpec(memory_space=pltpu.VMEM),
            pl.BlockSpec(memory_space=pltpu.VMEM),
            pl.BlockSpec(memory_space=pltpu.SMEM),
            pl.BlockSpec(memory_space=pltpu.SMEM),
        ],
        out_specs=pl.BlockSpec(memory_space=pltpu.VMEM),
        scratch_shapes=[
            pltpu.VMEM((N_DEV, m_blk, m_per), jnp.int8),
            pltpu.SemaphoreType.DMA((N_DEV,)),
            pltpu.SemaphoreType.DMA((N_DEV,)),
        ],
    )(x, w_mat, scale_x, scale_w)
